# baseline (speedup 1.0000x reference)
import functools

import jax
import jax.numpy as jnp
from jax import lax
from jax.experimental import pallas as pl
from jax.experimental.pallas import tpu as pltpu
from jax.experimental.pallas import tpu_sc as plsc

N_UNITS = 100000
K_SUB = 5
BATCH = 1024
N_REC = 4096

NC = 2
NS = 16
NW = NC * NS
R_PER_W = N_REC // NW
LANES = 16
RCHUNK = 8
NSTEP = R_PER_W // RCHUNK
GROWS = RCHUNK * K_SUB
BCHUNKS = BATCH // LANES


def _sc_body(et_hbm, idx_hbm, c_hbm, out_hbm,
             g0, g1, or0, or1, idxv, cv,
             sg0, sg1, so0, so1):
    cid = lax.axis_index("c")
    sid = lax.axis_index("s")
    wid = sid * NC + cid
    ebase = wid * (R_PER_W * K_SUB)
    rbase = wid * R_PER_W

    pltpu.sync_copy(idx_hbm.at[pl.ds(ebase, R_PER_W * K_SUB)], idxv)
    pltpu.sync_copy(c_hbm, cv)

    gbufs = (g0, g1)
    gsems = (sg0, sg1)
    orows = (or0, or1)
    osems = (so0, so1)

    def issue(step, par):
        pltpu.async_copy(
            et_hbm.at[idxv.at[pl.ds(step * GROWS, GROWS)]],
            gbufs[par],
            gsems[par],
        )

    def g_wait(par):
        pltpu.make_async_copy(
            et_hbm.at[pl.ds(0, GROWS)], gbufs[par], gsems[par]).wait()

    issue(0, 0)
    issue(1, 1)

    def step_body(sc, carry):
        for par in range(2):
            step = sc * 2 + par
            g_wait(par)
            gb = gbufs[par]
            orow = orows[par]

            @pl.when(step >= 2)
            def _wait_out():
                pltpu.make_async_copy(
                    orow, out_hbm.at[pl.ds(0, RCHUNK)], osems[par]).wait()

            def m_body(m, c2):
                o = m * LANES
                c5 = cv[pl.ds(o, LANES)] * jnp.float32(K_SUB)
                for rr in range(RCHUNK):
                    s = gb[rr * K_SUB, pl.ds(o, LANES)]
                    for k in range(1, K_SUB):
                        s = s + gb[rr * K_SUB + k, pl.ds(o, LANES)]
                    t = c5 - s
                    p = 1.0 / (1.0 + jnp.exp(-t))
                    orow[rr, pl.ds(o, LANES)] = p
                return c2

            lax.fori_loop(0, BCHUNKS, m_body, 0, unroll=2)

            @pl.when(step + 2 < NSTEP)
            def _refill():
                issue(step + 2, par)

            pltpu.async_copy(
                orow,
                out_hbm.at[pl.ds(rbase + step * RCHUNK, RCHUNK)],
                osems[par],
            )
        return carry

    lax.fori_loop(0, NSTEP // 2, step_body, 0)

    pltpu.make_async_copy(or0, out_hbm.at[pl.ds(0, RCHUNK)], so0).wait()
    pltpu.make_async_copy(or1, out_hbm.at[pl.ds(0, RCHUNK)], so1).wait()


@jax.jit
def _sc_call(et, idxf, conc):
    mesh = plsc.VectorSubcoreMesh(core_axis_name="c", subcore_axis_name="s")
    f = functools.partial(
        pl.kernel,
        out_type=jax.ShapeDtypeStruct((N_REC, BATCH), jnp.float32),
        mesh=mesh,
        compiler_params=pltpu.CompilerParams(
            needs_layout_passes=False, use_tc_tiling_on_sc=True),
        scratch_types=[
            pltpu.VMEM((GROWS, BATCH), jnp.float32),
            pltpu.VMEM((GROWS, BATCH), jnp.float32),
            pltpu.VMEM((RCHUNK, BATCH), jnp.float32),
            pltpu.VMEM((RCHUNK, BATCH), jnp.float32),
            pltpu.VMEM((R_PER_W * K_SUB,), jnp.int32),
            pltpu.VMEM((BATCH,), jnp.float32),
            pltpu.SemaphoreType.DMA,
            pltpu.SemaphoreType.DMA,
            pltpu.SemaphoreType.DMA,
            pltpu.SemaphoreType.DMA,
        ],
    )(_sc_body)
    return f(et, idxf, conc)


def kernel(energies, concentrations, receptor_indices):
    et = energies.T
    idxf = receptor_indices.astype(jnp.int32).reshape(-1)
    out_t = _sc_call(et, idxf, concentrations)
    return out_t.T


# speedup vs baseline: 1.0026x; 1.0026x over previous
import functools

import jax
import jax.numpy as jnp
from jax import lax
from jax.experimental import pallas as pl
from jax.experimental.pallas import tpu as pltpu
from jax.experimental.pallas import tpu_sc as plsc

N_UNITS = 100000
K_SUB = 5
BATCH = 1024
N_REC = 4096

NC = 2
NS = 16
NW = NC * NS
R_PER_W = N_REC // NW
LANES = 16
RCHUNK = 8
NSTEP = R_PER_W // RCHUNK
GROWS = RCHUNK * K_SUB
BCHUNKS = BATCH // LANES


def _sc_body(et_hbm, idx_hbm, c_hbm, out_hbm,
             gbuf, orow, idxv, cv, qsem, osem):
    cid = lax.axis_index("c")
    sid = lax.axis_index("s")
    wid = sid * NC + cid
    ebase = wid * (R_PER_W * K_SUB)
    rbase = wid * R_PER_W

    pltpu.sync_copy(idx_hbm.at[pl.ds(ebase, R_PER_W * K_SUB)], idxv)
    pltpu.sync_copy(c_hbm, cv)

    def issue(step):
        par = lax.rem(step, 2)
        pltpu.async_copy(
            et_hbm.at[idxv.at[pl.ds(step * GROWS, GROWS)]],
            gbuf.at[pl.ds(par * GROWS, GROWS)],
            qsem.at[par],
        )

    issue(0)
    issue(1)

    def step_body(step, carry):
        par = lax.rem(step, 2)
        pltpu.make_async_copy(
            et_hbm.at[pl.ds(0, GROWS)],
            gbuf.at[pl.ds(par * GROWS, GROWS)],
            qsem.at[par],
        ).wait()
        gb0 = par * GROWS
        ob0 = par * RCHUNK

        @pl.when(step >= 2)
        def _wait_out():
            pltpu.make_async_copy(
                orow.at[pl.ds(ob0, RCHUNK)],
                out_hbm.at[pl.ds(0, RCHUNK)],
                osem.at[par],
            ).wait()

        def m_body(m, c2):
            o = m * LANES
            c5 = cv[pl.ds(o, LANES)] * jnp.float32(K_SUB)
            for rr in range(RCHUNK):
                s = gbuf[gb0 + rr * K_SUB, pl.ds(o, LANES)]
                for k in range(1, K_SUB):
                    s = s + gbuf[gb0 + rr * K_SUB + k, pl.ds(o, LANES)]
                t = c5 - s
                p = 1.0 / (1.0 + jnp.exp(-t))
                orow[ob0 + rr, pl.ds(o, LANES)] = p
            return c2

        lax.fori_loop(0, BCHUNKS, m_body, 0)

        @pl.when(step + 2 < NSTEP)
        def _refill():
            issue(step + 2)

        pltpu.async_copy(
            orow.at[pl.ds(ob0, RCHUNK)],
            out_hbm.at[pl.ds(rbase + step * RCHUNK, RCHUNK)],
            osem.at[par],
        )
        return carry

    lax.fori_loop(0, NSTEP, step_body, 0)

    for par in range(2):
        pltpu.make_async_copy(
            orow.at[pl.ds(par * RCHUNK, RCHUNK)],
            out_hbm.at[pl.ds(0, RCHUNK)],
            osem.at[par],
        ).wait()


@jax.jit
def _sc_call(et, idxf, conc):
    mesh = plsc.VectorSubcoreMesh(core_axis_name="c", subcore_axis_name="s")
    f = functools.partial(
        pl.kernel,
        out_type=jax.ShapeDtypeStruct((N_REC, BATCH), jnp.float32),
        mesh=mesh,
        compiler_params=pltpu.CompilerParams(
            needs_layout_passes=False, use_tc_tiling_on_sc=True),
        scratch_types=[
            pltpu.VMEM((2 * GROWS, BATCH), jnp.float32),
            pltpu.VMEM((2 * RCHUNK, BATCH), jnp.float32),
            pltpu.VMEM((R_PER_W * K_SUB,), jnp.int32),
            pltpu.VMEM((BATCH,), jnp.float32),
            pltpu.SemaphoreType.DMA((2,)),
            pltpu.SemaphoreType.DMA((2,)),
        ],
    )(_sc_body)
    return f(et, idxf, conc)


def kernel(energies, concentrations, receptor_indices):
    et = energies.T
    idxf = receptor_indices.astype(jnp.int32).reshape(-1)
    out_t = _sc_call(et, idxf, concentrations)
    return out_t.T


# speedup vs baseline: 2.3358x; 2.3298x over previous
import functools

import jax
import jax.numpy as jnp
from jax import lax
from jax.experimental import pallas as pl
from jax.experimental.pallas import tpu as pltpu
from jax.experimental.pallas import tpu_sc as plsc

N_UNITS = 100000
K_SUB = 5
BATCH = 1024
N_REC = 4096

NC = 2
NS = 16
NW = NC * NS
R_PER_W = N_REC // NW
LANES = 16
RCHUNK = 8
NSTEP = R_PER_W // RCHUNK
GROWS = RCHUNK * K_SUB
BCHUNKS = BATCH // LANES


def _sc_body(et_hbm, idx_hbm, c_hbm, out_hbm,
             g0, g1, or0, or1, idxv, cv,
             sg0, sg1, so0, so1):
    cid = lax.axis_index("c")
    sid = lax.axis_index("s")
    wid = sid * NC + cid
    ebase = wid * (R_PER_W * K_SUB)
    rbase = wid * R_PER_W

    pltpu.sync_copy(idx_hbm.at[pl.ds(ebase, R_PER_W * K_SUB)], idxv)
    pltpu.sync_copy(c_hbm, cv)

    gbufs = (g0, g1)
    gsems = (sg0, sg1)
    orows = (or0, or1)
    osems = (so0, so1)

    def issue(step, par):
        pltpu.async_copy(
            et_hbm.at[idxv.at[pl.ds(step * GROWS, GROWS)]],
            gbufs[par],
            gsems[par],
        )

    def g_wait(par):
        pltpu.make_async_copy(
            et_hbm.at[pl.ds(0, GROWS)], gbufs[par], gsems[par]).wait()

    issue(0, 0)
    issue(1, 1)

    def step_body(sc, carry):
        for par in range(2):
            step = sc * 2 + par
            g_wait(par)
            gb = gbufs[par]
            orow = orows[par]

            @pl.when(step >= 2)
            def _wait_out():
                pltpu.make_async_copy(
                    orow, out_hbm.at[pl.ds(0, RCHUNK)], osems[par]).wait()

            def m_body(m, c2):
                o = m * LANES
                c5 = cv[pl.ds(o, LANES)] * jnp.float32(K_SUB)
                for rr in range(RCHUNK):
                    s = gb[rr * K_SUB, pl.ds(o, LANES)]
                    for k in range(1, K_SUB):
                        s = s + gb[rr * K_SUB + k, pl.ds(o, LANES)]
                    t = c5 - s
                    p = 1.0 / (1.0 + jnp.exp(-t))
                    orow[rr, pl.ds(o, LANES)] = p
                return c2

            lax.fori_loop(0, BCHUNKS, m_body, 0)

            @pl.when(step + 2 < NSTEP)
            def _refill():
                issue(step + 2, par)

            pltpu.async_copy(
                orow,
                out_hbm.at[pl.ds(rbase + step * RCHUNK, RCHUNK)],
                osems[par],
            )
        return carry

    lax.fori_loop(0, NSTEP // 2, step_body, 0)

    pltpu.make_async_copy(or0, out_hbm.at[pl.ds(0, RCHUNK)], so0).wait()
    pltpu.make_async_copy(or1, out_hbm.at[pl.ds(0, RCHUNK)], so1).wait()


@jax.jit
def _sc_call(et, idxf, conc):
    mesh = plsc.VectorSubcoreMesh(core_axis_name="c", subcore_axis_name="s")
    f = functools.partial(
        pl.kernel,
        out_type=jax.ShapeDtypeStruct((N_REC, BATCH), jnp.float32),
        mesh=mesh,
        compiler_params=pltpu.CompilerParams(
            needs_layout_passes=False, use_tc_tiling_on_sc=True),
        scratch_types=[
            pltpu.VMEM((GROWS, BATCH), jnp.float32),
            pltpu.VMEM((GROWS, BATCH), jnp.float32),
            pltpu.VMEM((RCHUNK, BATCH), jnp.float32),
            pltpu.VMEM((RCHUNK, BATCH), jnp.float32),
            pltpu.VMEM((R_PER_W * K_SUB,), jnp.int32),
            pltpu.VMEM((BATCH,), jnp.float32),
            pltpu.SemaphoreType.DMA,
            pltpu.SemaphoreType.DMA,
            pltpu.SemaphoreType.DMA,
            pltpu.SemaphoreType.DMA,
        ],
    )(_sc_body)
    return f(et, idxf, conc)


def kernel(energies, concentrations, receptor_indices):
    et = energies.T
    idxf = receptor_indices.astype(jnp.int32).reshape(-1)
    out_t = _sc_call(et, idxf, concentrations)
    return out_t.T
